# baseline (speedup 1.0000x reference)
import functools
import jax
import jax.numpy as jnp
from jax import lax
from jax.experimental import pallas as pl
from jax.experimental.pallas import tpu as pltpu, tpu_sc as plsc

_NC, _NS = 2, 16
_NW = _NC * _NS
_L = 16
_CH = 16 * 1024
_NQ = 4


def _sc_body(x_hbm, pe_hbm, out_hbm, pe_buf, xb0, xb1, sp,
             di0, di1, do0, do1, pe_sem, *, n_batch):
    cid = lax.axis_index("c")
    sid = lax.axis_index("s")
    wid = sid * _NC + cid
    pe0 = wid * (_NQ * _CH)
    xb = (xb0, xb1)
    din = (di0, di1)
    dout = (do0, do1)
    nch = _NQ * n_batch
    pending_out = [None, None]

    def slot(j):
        return sp.at[pl.ds((sid * 3 + j) * _CH, _CH)]

    def xoff(i):
        q, b = i // n_batch, i % n_batch
        return b * (_NW * _NQ * _CH) + pe0 + q * _CH

    def start_in(i):
        j = i % 2
        if pending_out[j] is not None:
            pending_out[j].wait()
            pending_out[j] = None
        return pltpu.async_copy(x_hbm.at[pl.ds(xoff(i), _CH)], slot(j), din[j])

    def start_pe(q):
        return pltpu.async_copy(pe_hbm.at[pl.ds(pe0 + q * _CH, _CH)],
                                slot(2), pe_sem)

    pe_load = start_pe(0)
    load = start_in(0)
    for i in range(nch):
        j = i % 2
        q = i // n_batch
        if i % n_batch == 0:
            pe_load.wait()
            pltpu.sync_copy(slot(2), pe_buf)
        if i + 1 < nch:
            nxt = start_in(i + 1)
            if (i + 1) % n_batch == 0:
                pe_load = start_pe(q + 1)
        else:
            nxt = None
        load.wait()
        pltpu.sync_copy(slot(j), xb[j])

        @plsc.parallel_loop(0, _CH, step=_L, unroll=8)
        def _(k):
            plsc.addupdate(xb[j].at[pl.ds(k, _L)], pe_buf[pl.ds(k, _L)])

        pltpu.sync_copy(xb[j], slot(j))
        pending_out[j] = pltpu.async_copy(slot(j),
                                          out_hbm.at[pl.ds(xoff(i), _CH)],
                                          dout[j])
        load = nxt
    for j in range(2):
        if pending_out[j] is not None:
            pending_out[j].wait()


def kernel(x, pe_table):
    B, S, D = x.shape
    mesh = plsc.VectorSubcoreMesh(core_axis_name="c", subcore_axis_name="s",
                                  num_cores=_NC, num_subcores=_NS)
    out = pl.kernel(
        functools.partial(_sc_body, n_batch=B),
        out_type=jax.ShapeDtypeStruct((B * S * D,), jnp.float32),
        mesh=mesh,
        scratch_types=[
            pltpu.VMEM((_CH,), jnp.float32),
            pltpu.VMEM((_CH,), jnp.float32),
            pltpu.VMEM((_CH,), jnp.float32),
            pltpu.VMEM_SHARED((_NS * 3 * _CH,), jnp.float32),
            pltpu.SemaphoreType.DMA,
            pltpu.SemaphoreType.DMA,
            pltpu.SemaphoreType.DMA,
            pltpu.SemaphoreType.DMA,
            pltpu.SemaphoreType.DMA,
        ],
    )(x.reshape(-1), pe_table.reshape(-1))
    return out.reshape(B, S, D)


# speedup vs baseline: 5.8470x; 5.8470x over previous
import jax
import jax.numpy as jnp
from jax.experimental import pallas as pl

_TS = 512


def _add_body(x_ref, pe_ref, o_ref):
    o_ref[0] = x_ref[0] + pe_ref[...]


def kernel(x, pe_table):
    B, S, D = x.shape
    grid = (S // _TS, B)
    return pl.pallas_call(
        _add_body,
        grid=grid,
        in_specs=[
            pl.BlockSpec((1, _TS, D), lambda i, b: (b, i, 0)),
            pl.BlockSpec((_TS, D), lambda i, b: (i, 0)),
        ],
        out_specs=pl.BlockSpec((1, _TS, D), lambda i, b: (b, i, 0)),
        out_shape=jax.ShapeDtypeStruct((B, S, D), x.dtype),
    )(x, pe_table[:S])
